# Initial kernel scaffold; baseline (speedup 1.0000x reference)
import jax
import jax.numpy as jnp
from jax.experimental import pallas as pl


def kernel(output, target):
    raise NotImplementedError("write your pallas kernel here")


# speedup vs baseline: 1.0662x; 1.0662x over previous
import jax
import jax.numpy as jnp
from jax import lax
from jax.experimental import pallas as pl
from jax.experimental.pallas import tpu as pltpu

_EPSILON = 0.3


def _stage1_body(x_ref, t_ref, m_ref, l_ref):
    x = x_ref[...]
    t = t_ref[...]
    cols = lax.broadcasted_iota(jnp.int32, x.shape, 1)
    is_t = cols == t
    ninf = jnp.float32(-jnp.inf)
    out_y = jnp.max(jnp.where(is_t, x, ninf), axis=1)
    mmax = jnp.max(jnp.where(is_t, ninf, x), axis=1)
    rmax = jnp.maximum(out_y, mmax)
    s = jnp.sum(jnp.exp(x - rmax[:, None]), axis=1)
    lse = rmax + jnp.log(s)
    margin = out_y - mmax
    ell = jnp.where(margin > 0, 1.0 - margin, 1.0 - out_y + lse)
    ell = jnp.maximum(ell, 0.0)
    m_ref[...] = margin[:, None]
    l_ref[...] = ell[:, None]


def _stage2_body(mj_ref, lj_ref, mrow_ref, o_ref, p1_ref, ns_ref):
    step = pl.program_id(0)
    nstep = pl.num_programs(0)
    ch = mj_ref.shape[0]
    b = mrow_ref.shape[1]

    mrow = mrow_ref[...]
    mj = mj_ref[...]
    lj = lj_ref[...]

    kidx = lax.broadcasted_iota(jnp.int32, (ch, b), 1)
    jidx = lax.broadcasted_iota(jnp.int32, (ch, b), 0) + step * ch
    lt = mrow < mj
    tie = (mrow == mj) & (kidx <= jidx)
    w = mrow + 1.0
    g = jnp.sum(jnp.where(lt | tie, w, 0.0), axis=1)

    n_neg = jnp.sum((mrow < 0.0).astype(jnp.float32))
    thr = jnp.floor((1.0 - _EPSILON) ** 2 * b + (1.0 - _EPSILON) * n_neg)

    cond = (g <= thr + 2.0).astype(jnp.float32)
    p1 = jnp.sum(cond * lj[:, 0])
    ns = jnp.sum(cond)

    @pl.when(step == 0)
    def _init():
        p1_ref[0, 0] = p1
        ns_ref[0, 0] = ns

    @pl.when(step > 0)
    def _acc():
        p1_ref[0, 0] += p1
        ns_ref[0, 0] += ns

    @pl.when(step == nstep - 1)
    def _fin():
        npcl1 = p1_ref[0, 0]
        npcl2 = thr - ns_ref[0, 0]
        res = jnp.where(npcl1 < npcl2, npcl1, npcl2)
        o_ref[...] = jnp.full((1, 1), res, dtype=jnp.float32)


def kernel(output, target):
    b, c = output.shape
    target = target.astype(jnp.int32).reshape(b, 1)

    br = 512
    margin_col, l_col = pl.pallas_call(
        _stage1_body,
        grid=(b // br,),
        in_specs=[
            pl.BlockSpec((br, c), lambda i: (i, 0)),
            pl.BlockSpec((br, 1), lambda i: (i, 0)),
        ],
        out_specs=[
            pl.BlockSpec((br, 1), lambda i: (i, 0)),
            pl.BlockSpec((br, 1), lambda i: (i, 0)),
        ],
        out_shape=[
            jax.ShapeDtypeStruct((b, 1), jnp.float32),
            jax.ShapeDtypeStruct((b, 1), jnp.float32),
        ],
    )(output, target)

    m_row = margin_col.reshape(1, b)

    ch = 256
    out = pl.pallas_call(
        _stage2_body,
        grid=(b // ch,),
        in_specs=[
            pl.BlockSpec((ch, 1), lambda i: (i, 0)),
            pl.BlockSpec((ch, 1), lambda i: (i, 0)),
            pl.BlockSpec((1, b), lambda i: (0, 0)),
        ],
        out_specs=pl.BlockSpec((1, 1), lambda i: (0, 0)),
        out_shape=jax.ShapeDtypeStruct((1, 1), jnp.float32),
        scratch_shapes=[
            pltpu.SMEM((1, 1), jnp.float32),
            pltpu.SMEM((1, 1), jnp.float32),
        ],
    )(margin_col, l_col, m_row)

    return out[0, 0]


# speedup vs baseline: 1.3590x; 1.2746x over previous
import jax
import jax.numpy as jnp
from jax import lax
from jax.experimental import pallas as pl
from jax.experimental.pallas import tpu as pltpu

_EPSILON = 0.3
_BR = 256


def _dotg(a, b):
    return lax.dot_general(a, b, (((0,), (0,)), ((), ())),
                           preferred_element_type=jnp.float32)


def _body(x_ref, t_ref, o_ref, mcol_ref, mrow_ref, lrow_ref, g_ref,
          eye_ref, nn_ref):
    s = pl.program_id(0)
    ns = pl.num_programs(0)
    br, c = x_ref.shape
    b = mcol_ref.shape[0]

    @pl.when(s == 0)
    def _init():
        r = lax.broadcasted_iota(jnp.int32, (br, br), 0)
        q = lax.broadcasted_iota(jnp.int32, (br, br), 1)
        eye_ref[...] = (r == q).astype(jnp.float32)
        g_ref[...] = jnp.zeros_like(g_ref)
        nn_ref[0, 0] = 0.0

    x = x_ref[...]
    t = t_ref[...]
    cols = lax.broadcasted_iota(jnp.int32, (br, c), 1)
    is_t = cols == t
    ninf = jnp.float32(-jnp.inf)
    out_y = jnp.max(jnp.where(is_t, x, ninf), axis=1, keepdims=True)
    mmax = jnp.max(jnp.where(is_t, ninf, x), axis=1, keepdims=True)
    rmax = jnp.maximum(out_y, mmax)
    ssum = jnp.sum(jnp.exp(x - rmax), axis=1, keepdims=True)
    lse = rmax + jnp.log(ssum)
    margin = out_y - mmax
    ell = jnp.where(margin > 0, 1.0 - margin, 1.0 - out_y + lse)
    ell = jnp.maximum(ell, 0.0)

    base = s * br
    mcol_ref[pl.ds(base, br), :] = margin
    eye = eye_ref[...]
    mrow = _dotg(margin, eye)
    lrow = _dotg(ell, eye)
    mrow_ref[:, pl.ds(base, br)] = mrow
    lrow_ref[:, pl.ds(base, br)] = lrow
    nn_ref[0, 0] += jnp.sum((margin < 0.0).astype(jnp.float32))

    w_blk = margin + 1.0

    mcol_all = mcol_ref[...]
    kcol = lax.broadcasted_iota(jnp.int32, (b, 1), 0)
    w_all = jnp.where(kcol < base, mcol_all + 1.0, 0.0)
    lef = (mcol_all <= mrow).astype(jnp.float32)
    g_a1 = _dotg(w_all, lef)

    kd = lax.broadcasted_iota(jnp.int32, (br, br), 0)
    jd = lax.broadcasted_iota(jnp.int32, (br, br), 1)
    ltd = margin < mrow
    tied = (margin == mrow) & (kd <= jd)
    df = (ltd | tied).astype(jnp.float32)
    g_a2 = _dotg(w_blk, df)

    g_ref[:, pl.ds(base, br)] += g_a1 + g_a2

    mrow_all = mrow_ref[...]
    ltf = (margin < mrow_all).astype(jnp.float32)
    g_b = _dotg(w_blk, ltf)
    jrow = lax.broadcasted_iota(jnp.int32, (1, b), 1)
    g_ref[...] += jnp.where(jrow < base, g_b, 0.0)

    @pl.when(s == ns - 1)
    def _fin():
        n_neg = nn_ref[0, 0]
        thr = jnp.floor((1.0 - _EPSILON) ** 2 * b + (1.0 - _EPSILON) * n_neg)
        cond = (g_ref[...] <= thr + 2.0).astype(jnp.float32)
        p1 = jnp.sum(cond * lrow_ref[...])
        nsel = jnp.sum(cond)
        p2 = thr - nsel
        o_ref[...] = jnp.full((1, 1), jnp.where(p1 < p2, p1, p2),
                              dtype=jnp.float32)


def kernel(output, target):
    b, c = output.shape
    target = target.astype(jnp.int32).reshape(b, 1)

    out = pl.pallas_call(
        _body,
        grid=(b // _BR,),
        in_specs=[
            pl.BlockSpec((_BR, c), lambda i: (i, 0)),
            pl.BlockSpec((_BR, 1), lambda i: (i, 0)),
        ],
        out_specs=pl.BlockSpec((1, 1), lambda i: (0, 0)),
        out_shape=jax.ShapeDtypeStruct((1, 1), jnp.float32),
        scratch_shapes=[
            pltpu.VMEM((b, 1), jnp.float32),
            pltpu.VMEM((1, b), jnp.float32),
            pltpu.VMEM((1, b), jnp.float32),
            pltpu.VMEM((1, b), jnp.float32),
            pltpu.VMEM((_BR, _BR), jnp.float32),
            pltpu.SMEM((1, 1), jnp.float32),
        ],
    )(output, target)

    return out[0, 0]


# speedup vs baseline: 1.3941x; 1.0259x over previous
import jax
import jax.numpy as jnp
from jax import lax
from jax.experimental import pallas as pl
from jax.experimental.pallas import tpu as pltpu

_EPSILON = 0.3
_BR = 256


def _dot00(a, b):
    return lax.dot_general(a, b, (((0,), (0,)), ((), ())),
                           preferred_element_type=jnp.float32)


def _dot11(a, b):
    return lax.dot_general(a, b, (((1,), (1,)), ((), ())),
                           preferred_element_type=jnp.float32)


def _body(x_ref, t_ref, cols_ref, o_ref, mrow_ref, lrow_ref, g_ref,
          eye_ref, nn_ref):
    s = pl.program_id(0)
    ns = pl.num_programs(0)
    br, c = x_ref.shape
    b = mrow_ref.shape[1]

    @pl.when(s == 0)
    def _init():
        r = lax.broadcasted_iota(jnp.int32, (br, br), 0)
        q = lax.broadcasted_iota(jnp.int32, (br, br), 1)
        eye_ref[...] = (r == q).astype(jnp.float32)
        g_ref[...] = jnp.zeros_like(g_ref)
        nn_ref[0, 0] = 0.0

    x = x_ref[...]
    t = t_ref[...]
    is_t = cols_ref[...] == t
    ninf = jnp.float32(-jnp.inf)
    out_y = jnp.max(jnp.where(is_t, x, ninf), axis=1, keepdims=True)
    mmax = jnp.max(jnp.where(is_t, ninf, x), axis=1, keepdims=True)
    rmax = jnp.maximum(out_y, mmax)
    ssum = jnp.sum(jnp.exp(x - rmax), axis=1, keepdims=True)
    lse = rmax + jnp.log(ssum)
    margin = out_y - mmax
    ell = jnp.where(margin > 0, 1.0 - margin, 1.0 - out_y + lse)
    ell = jnp.maximum(ell, 0.0)

    base = s * br
    eye = eye_ref[...]
    mrow = _dot00(margin, eye)
    lrow = _dot00(ell, eye)
    mrow_ref[:, pl.ds(base, br)] = mrow
    lrow_ref[:, pl.ds(base, br)] = lrow
    nn_ref[0, 0] += jnp.sum((margin < 0.0).astype(jnp.float32))

    w_blk = margin + 1.0
    mrow_all = mrow_ref[...]

    ltf = (margin < mrow_all).astype(jnp.float32)

    g_b = _dot00(w_blk, ltf)
    jrow = lax.broadcasted_iota(jnp.int32, (1, b), 1)
    g_ref[...] += jnp.where(jrow < base, g_b, 0.0)

    w_kill = jnp.where(jrow < base, mrow_all + 1.0, 0.0)
    wtot = jnp.sum(w_kill)
    d_col = _dot11(ltf, w_kill)
    g_a1 = wtot - d_col

    kd = lax.broadcasted_iota(jnp.int32, (br, br), 1)
    jd = lax.broadcasted_iota(jnp.int32, (br, br), 0)
    ltd = mrow < margin
    tied = (mrow == margin) & (kd <= jd)
    df = (ltd | tied).astype(jnp.float32)
    g_a2 = _dot11(df, mrow + 1.0)

    g_row = _dot00(g_a1 + g_a2, eye)
    g_ref[:, pl.ds(base, br)] += g_row

    @pl.when(s == ns - 1)
    def _fin():
        n_neg = nn_ref[0, 0]
        thr = jnp.floor((1.0 - _EPSILON) ** 2 * b + (1.0 - _EPSILON) * n_neg)
        cond = (g_ref[...] <= thr + 2.0).astype(jnp.float32)
        p1 = jnp.sum(cond * lrow_ref[...])
        nsel = jnp.sum(cond)
        p2 = thr - nsel
        o_ref[...] = jnp.full((1, 1), jnp.where(p1 < p2, p1, p2),
                              dtype=jnp.float32)


def kernel(output, target):
    b, c = output.shape
    target = target.astype(jnp.int32).reshape(b, 1)
    cols = jnp.arange(c, dtype=jnp.int32).reshape(1, c)

    out = pl.pallas_call(
        _body,
        grid=(b // _BR,),
        in_specs=[
            pl.BlockSpec((_BR, c), lambda i: (i, 0)),
            pl.BlockSpec((_BR, 1), lambda i: (i, 0)),
            pl.BlockSpec((1, c), lambda i: (0, 0)),
        ],
        out_specs=pl.BlockSpec((1, 1), lambda i: (0, 0)),
        out_shape=jax.ShapeDtypeStruct((1, 1), jnp.float32),
        scratch_shapes=[
            pltpu.VMEM((1, b), jnp.float32),
            pltpu.VMEM((1, b), jnp.float32),
            pltpu.VMEM((1, b), jnp.float32),
            pltpu.VMEM((_BR, _BR), jnp.float32),
            pltpu.SMEM((1, 1), jnp.float32),
        ],
    )(output, target, cols)

    return out[0, 0]
